# baseline (speedup 1.0000x reference)
import dataclasses
import functools

import jax
import jax.numpy as jnp
from jax import lax
from jax.experimental import pallas as pl
from jax.experimental.pallas import tpu as pltpu
from jax.experimental.pallas import tpu_sc as plsc

_N = 10000
_NPAD = 10240
_NPART = _NPAD // 16
_E = 320000
_NW = 32
_EPW = _E // _NW
_B = 2000
_NC = _EPW // _B

_mesh = plsc.VectorSubcoreMesh(core_axis_name="c", subcore_axis_name="s")
_CP = pltpu.CompilerParams()
if "needs_layout_passes" in pltpu.CompilerParams.__dataclass_fields__:
    _CP = dataclasses.replace(_CP, needs_layout_passes=False)
_CP = dataclasses.replace(_CP, use_tc_tiling_on_sc=False)


def _rsqrt16(x):
    i = plsc.bitcast(x, jnp.int32)
    i = jnp.int32(0x5F3759DF) - lax.shift_right_logical(i, 1)
    y = plsc.bitcast(i, jnp.float32)
    for _ in range(4):
        y = y * (1.5 - 0.5 * x * y * y)
    return y


_BK = 2000
_RPW = 20000 // _BK


@functools.partial(
    pl.kernel,
    out_type=[jax.ShapeDtypeStruct((_E,), jnp.float32),
              jax.ShapeDtypeStruct((_NPAD,), jnp.float32),
              jax.ShapeDtypeStruct((_NPAD,), jnp.float32)],
    mesh=_mesh,
    compiler_params=_CP,
    scratch_types=[
        pltpu.VMEM((_RPW, _BK), jnp.int32),
        pltpu.VMEM((_RPW, _BK), jnp.int32),
        pltpu.VMEM((_RPW * _BK,), jnp.float32),
        pltpu.VMEM((_EPW,), jnp.float32),
        pltpu.VMEM((_BK,), jnp.float32),
        pltpu.VMEM((_NPAD,), jnp.float32),
        pltpu.VMEM((_NPAD,), jnp.float32),
        pltpu.VMEM_SHARED((_NPAD,), jnp.float32),
        pltpu.VMEM_SHARED((_NPAD,), jnp.float32),
        pltpu.VMEM_SHARED((_NPAD,), jnp.float32),
        pltpu.VMEM_SHARED((_NPAD,), jnp.float32),
        pltpu.SemaphoreType.DMA,
        pltpu.SemaphoreType.DMA,
        pltpu.SemaphoreType.DMA,
        pltpu.SemaphoreType.DMA,
    ],
)
def _norm_kernel(src_hbm, dst_hbm, w_hbm, cw_hbm, dois_hbm, diis_hbm,
                 srcs, dsts, w_v, cw_v, ones_v, ta_v, tb_v,
                 wdo_sh, wdi_sh, cno_sh, cni_sh, m0, m1, m2, m3):
    c = lax.axis_index("c")
    s = lax.axis_index("s")
    wid = s * 2 + c

    pltpu.sync_copy(src_hbm.at[pl.ds(s * _RPW, _RPW)], srcs)
    pltpu.sync_copy(dst_hbm.at[pl.ds(s * _RPW, _RPW)], dsts)
    pltpu.sync_copy(w_hbm.at[pl.ds(s * _RPW * _BK, _RPW * _BK)], w_v)

    @pl.loop(0, _BK, step=16)
    def _(i):
        ones_v[pl.ds(i, 16)] = jnp.ones((16,), jnp.float32)

    @pl.loop(0, _NPART, step=16)
    def _(i):
        ta_v[pl.ds(i, 16)] = jnp.zeros((16,), jnp.float32)

    for tab in (wdo_sh, wdi_sh, cno_sh, cni_sh):
        pltpu.sync_copy(ta_v.at[pl.ds(0, _NPART)],
                        tab.at[pl.ds(s * _NPART, _NPART)])

    @pl.loop(0, _RPW * _BK, step=16)
    def _(i):
        w_v[pl.ds(i, 16)] = jnp.abs(w_v[pl.ds(i, 16)])

    plsc.subcore_barrier()

    sems = (m0, m1, m2, m3)
    pend = []
    for k in range(_RPW):
        if k >= 2:
            for dsc in pend[0]:
                dsc.wait()
            pend.pop(0)
        wslice = w_v.at[pl.ds(k * _BK, _BK)]
        pend.append((
            pltpu.async_copy(wslice, wdo_sh.at[srcs.at[k]], m0, add=True),
            pltpu.async_copy(wslice, wdi_sh.at[dsts.at[k]], m1, add=True),
            pltpu.async_copy(ones_v, cno_sh.at[srcs.at[k]], m2, add=True),
            pltpu.async_copy(ones_v, cni_sh.at[dsts.at[k]], m3, add=True),
        ))
    for group in pend:
        for dsc in group:
            dsc.wait()
    plsc.subcore_barrier()

    @pl.when(c == 0)
    def _():
        pltpu.sync_copy(cno_sh.at[pl.ds(s * _NPART, _NPART)],
                        ta_v.at[pl.ds(0, _NPART)])
        pltpu.sync_copy(cni_sh.at[pl.ds(s * _NPART, _NPART)],
                        tb_v.at[pl.ds(0, _NPART)])

        @pl.loop(0, _NPART, step=16)
        def _(i):
            ta_v[pl.ds(i, 16)] = _rsqrt16(
                jnp.maximum(ta_v[pl.ds(i, 16)], 1.0))
            tb_v[pl.ds(i, 16)] = _rsqrt16(
                jnp.maximum(tb_v[pl.ds(i, 16)], 1.0))

        pltpu.sync_copy(ta_v.at[pl.ds(0, _NPART)],
                        dois_hbm.at[pl.ds(s * _NPART, _NPART)])
        pltpu.sync_copy(tb_v.at[pl.ds(0, _NPART)],
                        diis_hbm.at[pl.ds(s * _NPART, _NPART)])

    pltpu.sync_copy(wdo_sh, ta_v)
    pltpu.sync_copy(wdi_sh, tb_v)
    for r in range(_EPW // _BK):
        row = c * (_EPW // _BK) + r

        @pl.loop(0, _BK // 16)
        def _(g):
            sv = srcs[row, pl.ds(g * 16, 16)]
            dv = dsts[row, pl.ds(g * 16, 16)]
            a = plsc.load_gather(ta_v, [sv])
            b = plsc.load_gather(tb_v, [dv])
            p = jnp.maximum(a * b, jnp.float32(1e-12))
            wv = w_v[pl.ds(c * _EPW + r * _BK + g * 16, 16)]
            cw_v[pl.ds(r * _BK + g * 16, 16)] = jnp.abs(wv) * _rsqrt16(p)

    pltpu.sync_copy(cw_v, cw_hbm.at[pl.ds(wid * _EPW, _EPW)])


_BE = 1000
_NCE = _EPW // _BE


def _make_edge_kernel(D):
    @functools.partial(
        pl.kernel,
        out_type=jax.ShapeDtypeStruct((2, _NPAD, D), jnp.float32),
        mesh=_mesh,
        compiler_params=_CP,
        scratch_types=[
            pltpu.VMEM((_NCE, _BE), jnp.int32),
            pltpu.VMEM((_NCE, _BE), jnp.int32),
            pltpu.VMEM((_EPW,), jnp.float32),
            pltpu.VMEM((_BE, D), jnp.float32),
            pltpu.VMEM((_BE, D), jnp.float32),
            pltpu.VMEM_SHARED((_NPAD, D), jnp.float32),
            pltpu.SemaphoreType.DMA,
            pltpu.SemaphoreType.DMA,
            pltpu.SemaphoreType.DMA,
            pltpu.SemaphoreType.DMA,
        ],
    )
    def k(h_hbm, src2_hbm, dst2_hbm, cw_hbm, out_hbm,
          srcs, dsts, cws, r0, r1, agg_sh, g0, g1, q0, q1):
        c = lax.axis_index("c")
        s = lax.axis_index("s")
        wid = s * 2 + c
        rows = (r0, r1)
        gsem = (g0, g1)
        ssem = (q0, q1)

        crow = wid * _NCE
        pltpu.sync_copy(src2_hbm.at[pl.ds(crow, _NCE)], srcs)
        pltpu.sync_copy(dst2_hbm.at[pl.ds(crow, _NCE)], dsts)
        pltpu.sync_copy(cw_hbm.at[pl.ds(wid * _EPW, _EPW)], cws)

        @pl.loop(0, _NPART)
        def _(i):
            for j in range(0, D, 16):
                r0[i, pl.ds(j, 16)] = jnp.zeros((16,), jnp.float32)

        pltpu.sync_copy(r0.at[pl.ds(0, _NPART)],
                        agg_sh.at[pl.ds(s * _NPART, _NPART)])
        plsc.subcore_barrier()

        gat = [None, None]
        scat = [None, None]
        gat[0] = pltpu.async_copy(h_hbm.at[srcs.at[0]], rows[0], gsem[0])
        for it in range(_NCE):
            b = it % 2
            nb = 1 - b
            if it + 1 < _NCE:
                if scat[nb] is not None:
                    scat[nb].wait()
                gat[nb] = pltpu.async_copy(h_hbm.at[srcs.at[it + 1]],
                                           rows[nb], gsem[nb])
            gat[b].wait()
            cbase = it * _BE

            @plsc.parallel_loop(0, _BE, step=1, unroll=8)
            def _(e):
                cvec = plsc.load_gather(
                    cws, [jnp.full((16,), cbase, jnp.int32) + e])
                for j in range(0, D, 16):
                    rows[b][e, pl.ds(j, 16)] = rows[b][e, pl.ds(j, 16)] * cvec

            scat[b] = pltpu.async_copy(rows[b], agg_sh.at[dsts.at[it]],
                                       ssem[b], add=True)

        for dsc in scat:
            if dsc is not None:
                dsc.wait()
        plsc.subcore_barrier()
        pltpu.sync_copy(agg_sh.at[pl.ds(s * _NPART, _NPART)],
                        out_hbm.at[c].at[pl.ds(s * _NPART, _NPART)])

    return k


_edge32 = _make_edge_kernel(32)
_edge16 = _make_edge_kernel(16)


def _h1_body(x_ref, w_ref, dois_ref, out_ref):
    h = jnp.dot(x_ref[...], w_ref[...], preferred_element_type=jnp.float32)
    d = dois_ref[...][:_N]
    out_ref[0:_N, :] = h * d[:, None]
    out_ref[_N:_NPAD, :] = jnp.zeros((_NPAD - _N, h.shape[1]), jnp.float32)


def _bn_elu(parts, diis, b, g, be):
    agg = parts[0, :_N] + parts[1, :_N]
    agg = agg * diis[:_N, None] + b[None, :]
    mu = jnp.mean(agg, axis=0, keepdims=True)
    var = jnp.mean((agg - mu) ** 2, axis=0, keepdims=True)
    xn = (agg - mu) * lax.rsqrt(var + 1e-5) * g[None, :] + be[None, :]
    return jnp.where(xn > 0, xn, jnp.exp(xn) - 1.0)


def _mid_body(parts_ref, diis_ref, b_ref, g_ref, be_ref, w_ref, dois_ref,
              out_ref):
    e = _bn_elu(parts_ref[...], diis_ref[...], b_ref[...], g_ref[...],
                be_ref[...])
    h = jnp.dot(e, w_ref[...], preferred_element_type=jnp.float32)
    h = h * dois_ref[...][:_N, None]
    out_ref[0:_N, :] = h
    out_ref[_N:_NPAD, :] = jnp.zeros((_NPAD - _N, h.shape[1]), jnp.float32)


def _efsum_body(ef_ref, out_ref):
    i = pl.program_id(0)

    @pl.when(i == 0)
    def _():
        out_ref[...] = jnp.zeros_like(out_ref)

    out_ref[...] += jnp.sum(ef_ref[...], axis=0, keepdims=True)


def _final_body(parts_ref, diis_ref, b_ref, g_ref, be_ref, em_ref, wc_ref,
                bc_ref, out_ref):
    x4 = _bn_elu(parts_ref[...], diis_ref[...], b_ref[...], g_ref[...],
                 be_ref[...])
    pooled = jnp.sum(x4, axis=0, keepdims=True)
    em = em_ref[...] * jnp.float32(1.0 / _E)
    cat = jnp.concatenate([pooled, em], axis=1)
    out_ref[...] = (jnp.dot(cat, wc_ref[...],
                            preferred_element_type=jnp.float32)
                    + bc_ref[...][None, :])


def _f32(*shape):
    return jax.ShapeDtypeStruct(shape, jnp.float32)


def kernel(node_features, edge_index, edge_features, edge_weights,
           W1, b1, gamma1, beta1, W2, b2, gamma2, beta2,
           W3, b3, gamma3, beta3, W4, b4, gamma4, beta4, Wc, bc):
    src = edge_index[0]
    dst = edge_index[1]
    src2 = src.reshape(_E // _BE, _BE)
    dst2 = dst.reshape(_E // _BE, _BE)
    src2k = src.reshape(_E // _BK, _BK)
    dst2k = dst.reshape(_E // _BK, _BK)

    cw, dois, diis = _norm_kernel(src2k, dst2k, edge_weights)

    h = pl.pallas_call(_h1_body, out_shape=_f32(_NPAD, 32))(
        node_features, W1, dois)

    layer_params = [(b1, gamma1, beta1, W2), (b2, gamma2, beta2, W3),
                    (b3, gamma3, beta3, W4)]
    for b, g, be, Wn in layer_params:
        aggp = _edge32(h, src2, dst2, cw)
        h = pl.pallas_call(_mid_body, out_shape=_f32(_NPAD, Wn.shape[1]))(
            aggp, diis, b, g, be, Wn, dois)

    aggp4 = _edge16(h, src2, dst2, cw)
    efsum = pl.pallas_call(
        _efsum_body,
        grid=(20,),
        in_specs=[pl.BlockSpec((_E // 20, 16), lambda i: (i, 0))],
        out_specs=pl.BlockSpec((1, 16), lambda i: (0, 0)),
        out_shape=_f32(1, 16),
    )(edge_features)
    out = pl.pallas_call(_final_body, out_shape=_f32(1, 10))(
        aggp4, diis, b4, gamma4, beta4, efsum, Wc, bc)
    return out


# speedup vs baseline: 1.0480x; 1.0480x over previous
import dataclasses
import functools

import jax
import jax.numpy as jnp
from jax import lax
from jax.experimental import pallas as pl
from jax.experimental.pallas import tpu as pltpu
from jax.experimental.pallas import tpu_sc as plsc

_N = 10000
_NPAD = 10240
_NPART = _NPAD // 16
_E = 320000
_NW = 32
_EPW = _E // _NW
_B = 2000
_NC = _EPW // _B

_mesh = plsc.VectorSubcoreMesh(core_axis_name="c", subcore_axis_name="s")
_CP = pltpu.CompilerParams()
if "needs_layout_passes" in pltpu.CompilerParams.__dataclass_fields__:
    _CP = dataclasses.replace(_CP, needs_layout_passes=False)
_CP = dataclasses.replace(_CP, use_tc_tiling_on_sc=False)


def _rsqrt16(x):
    i = plsc.bitcast(x, jnp.int32)
    i = jnp.int32(0x5F3759DF) - lax.shift_right_logical(i, 1)
    y = plsc.bitcast(i, jnp.float32)
    for _ in range(4):
        y = y * (1.5 - 0.5 * x * y * y)
    return y


_BK = 2000
_RPW = 20000 // _BK


@functools.partial(
    pl.kernel,
    out_type=[jax.ShapeDtypeStruct((_E,), jnp.float32),
              jax.ShapeDtypeStruct((_NPAD,), jnp.float32),
              jax.ShapeDtypeStruct((_NPAD,), jnp.float32)],
    mesh=_mesh,
    compiler_params=_CP,
    scratch_types=[
        pltpu.VMEM((_RPW * _BK,), jnp.int32),
        pltpu.VMEM((_RPW * _BK,), jnp.int32),
        pltpu.VMEM((_RPW * _BK,), jnp.float32),
        pltpu.VMEM((_EPW,), jnp.float32),
        pltpu.VMEM((_BK,), jnp.float32),
        pltpu.VMEM((_NPAD,), jnp.float32),
        pltpu.VMEM((_NPAD,), jnp.float32),
        pltpu.VMEM_SHARED((_NPAD,), jnp.float32),
        pltpu.VMEM_SHARED((_NPAD,), jnp.float32),
        pltpu.VMEM_SHARED((_NPAD,), jnp.float32),
        pltpu.VMEM_SHARED((_NPAD,), jnp.float32),
        pltpu.SemaphoreType.DMA,
        pltpu.SemaphoreType.DMA,
        pltpu.SemaphoreType.DMA,
        pltpu.SemaphoreType.DMA,
    ],
)
def _norm_kernel(src_hbm, dst_hbm, w_hbm, cw_hbm, dois_hbm, diis_hbm,
                 srcs, dsts, w_v, cw_v, ones_v, ta_v, tb_v,
                 wdo_sh, wdi_sh, cno_sh, cni_sh, m0, m1, m2, m3):
    c = lax.axis_index("c")
    s = lax.axis_index("s")
    wid = s * 2 + c

    pltpu.sync_copy(src_hbm.at[pl.ds(s * _RPW * _BK, _RPW * _BK)], srcs)
    pltpu.sync_copy(dst_hbm.at[pl.ds(s * _RPW * _BK, _RPW * _BK)], dsts)
    pltpu.sync_copy(w_hbm.at[pl.ds(s * _RPW * _BK, _RPW * _BK)], w_v)

    @pl.loop(0, _BK, step=16)
    def _(i):
        ones_v[pl.ds(i, 16)] = jnp.ones((16,), jnp.float32)

    @pl.loop(0, _NPART, step=16)
    def _(i):
        ta_v[pl.ds(i, 16)] = jnp.zeros((16,), jnp.float32)

    for tab in (wdo_sh, wdi_sh, cno_sh, cni_sh):
        pltpu.sync_copy(ta_v.at[pl.ds(0, _NPART)],
                        tab.at[pl.ds(s * _NPART, _NPART)])

    @pl.loop(0, _RPW * _BK, step=16)
    def _(i):
        w_v[pl.ds(i, 16)] = jnp.abs(w_v[pl.ds(i, 16)])

    plsc.subcore_barrier()

    sems = (m0, m1, m2, m3)
    pend = []
    for k in range(_RPW):
        if k >= 2:
            for dsc in pend[0]:
                dsc.wait()
            pend.pop(0)
        wslice = w_v.at[pl.ds(k * _BK, _BK)]
        sslice = srcs.at[pl.ds(k * _BK, _BK)]
        dslice = dsts.at[pl.ds(k * _BK, _BK)]
        pend.append((
            pltpu.async_copy(wslice, wdo_sh.at[sslice], m0, add=True),
            pltpu.async_copy(wslice, wdi_sh.at[dslice], m1, add=True),
            pltpu.async_copy(ones_v, cno_sh.at[sslice], m2, add=True),
            pltpu.async_copy(ones_v, cni_sh.at[dslice], m3, add=True),
        ))
    for group in pend:
        for dsc in group:
            dsc.wait()
    plsc.subcore_barrier()

    @pl.when(c == 0)
    def _():
        pltpu.sync_copy(cno_sh.at[pl.ds(s * _NPART, _NPART)],
                        ta_v.at[pl.ds(0, _NPART)])
        pltpu.sync_copy(cni_sh.at[pl.ds(s * _NPART, _NPART)],
                        tb_v.at[pl.ds(0, _NPART)])

        @pl.loop(0, _NPART, step=16)
        def _(i):
            ta_v[pl.ds(i, 16)] = _rsqrt16(
                jnp.maximum(ta_v[pl.ds(i, 16)], 1.0))
            tb_v[pl.ds(i, 16)] = _rsqrt16(
                jnp.maximum(tb_v[pl.ds(i, 16)], 1.0))

        pltpu.sync_copy(ta_v.at[pl.ds(0, _NPART)],
                        dois_hbm.at[pl.ds(s * _NPART, _NPART)])
        pltpu.sync_copy(tb_v.at[pl.ds(0, _NPART)],
                        diis_hbm.at[pl.ds(s * _NPART, _NPART)])

    pltpu.sync_copy(wdo_sh, ta_v)
    pltpu.sync_copy(wdi_sh, tb_v)
    for r in range(_EPW // _BK):

        @pl.loop(0, _BK // 16)
        def _(g):
            loc = c * _EPW + r * _BK + g * 16
            sv = srcs[pl.ds(loc, 16)]
            dv = dsts[pl.ds(loc, 16)]
            a = plsc.load_gather(ta_v, [sv])
            b = plsc.load_gather(tb_v, [dv])
            p = jnp.maximum(a * b, jnp.float32(1e-12))
            wv = w_v[pl.ds(loc, 16)]
            cw_v[pl.ds(r * _BK + g * 16, 16)] = jnp.abs(wv) * _rsqrt16(p)

    pltpu.sync_copy(cw_v, cw_hbm.at[pl.ds(wid * _EPW, _EPW)])


_BE = 1000
_NCE = _EPW // _BE


def _make_edge_kernel(D):
    @functools.partial(
        pl.kernel,
        out_type=jax.ShapeDtypeStruct((2, _NPAD, D), jnp.float32),
        mesh=_mesh,
        compiler_params=_CP,
        scratch_types=[
            pltpu.VMEM((_EPW,), jnp.int32),
            pltpu.VMEM((_EPW,), jnp.int32),
            pltpu.VMEM((_EPW,), jnp.float32),
            pltpu.VMEM((_BE, D), jnp.float32),
            pltpu.VMEM((_BE, D), jnp.float32),
            pltpu.VMEM_SHARED((_NPAD, D), jnp.float32),
            pltpu.SemaphoreType.DMA,
            pltpu.SemaphoreType.DMA,
            pltpu.SemaphoreType.DMA,
            pltpu.SemaphoreType.DMA,
        ],
    )
    def k(h_hbm, src2_hbm, dst2_hbm, cw_hbm, out_hbm,
          srcs, dsts, cws, r0, r1, agg_sh, g0, g1, q0, q1):
        c = lax.axis_index("c")
        s = lax.axis_index("s")
        wid = s * 2 + c
        rows = (r0, r1)
        gsem = (g0, g1)
        ssem = (q0, q1)

        base = wid * _EPW
        pltpu.sync_copy(src2_hbm.at[pl.ds(base, _EPW)], srcs)
        pltpu.sync_copy(dst2_hbm.at[pl.ds(base, _EPW)], dsts)
        pltpu.sync_copy(cw_hbm.at[pl.ds(base, _EPW)], cws)

        @pl.loop(0, _NPART)
        def _(i):
            for j in range(0, D, 16):
                r0[i, pl.ds(j, 16)] = jnp.zeros((16,), jnp.float32)

        pltpu.sync_copy(r0.at[pl.ds(0, _NPART)],
                        agg_sh.at[pl.ds(s * _NPART, _NPART)])
        plsc.subcore_barrier()

        gat = [None, None]
        scat = [None, None]
        gat[0] = pltpu.async_copy(h_hbm.at[srcs.at[pl.ds(0, _BE)]], rows[0],
                                  gsem[0])
        for it in range(_NCE):
            b = it % 2
            nb = 1 - b
            if it + 1 < _NCE:
                if scat[nb] is not None:
                    scat[nb].wait()
                gat[nb] = pltpu.async_copy(
                    h_hbm.at[srcs.at[pl.ds((it + 1) * _BE, _BE)]],
                    rows[nb], gsem[nb])
            gat[b].wait()
            cbase = it * _BE

            @plsc.parallel_loop(0, _BE, step=1, unroll=8)
            def _(e):
                cvec = plsc.load_gather(
                    cws, [jnp.full((16,), cbase, jnp.int32) + e])
                for j in range(0, D, 16):
                    rows[b][e, pl.ds(j, 16)] = rows[b][e, pl.ds(j, 16)] * cvec

            scat[b] = pltpu.async_copy(
                rows[b], agg_sh.at[dsts.at[pl.ds(it * _BE, _BE)]],
                ssem[b], add=True)

        for dsc in scat:
            if dsc is not None:
                dsc.wait()
        plsc.subcore_barrier()
        pltpu.sync_copy(agg_sh.at[pl.ds(s * _NPART, _NPART)],
                        out_hbm.at[c].at[pl.ds(s * _NPART, _NPART)])

    return k


_edge32 = _make_edge_kernel(32)
_edge16 = _make_edge_kernel(16)


def _h1_body(x_ref, w_ref, dois_ref, out_ref):
    h = jnp.dot(x_ref[...], w_ref[...], preferred_element_type=jnp.float32)
    d = dois_ref[...][:_N]
    out_ref[0:_N, :] = h * d[:, None]
    out_ref[_N:_NPAD, :] = jnp.zeros((_NPAD - _N, h.shape[1]), jnp.float32)


def _bn_elu(parts, diis, b, g, be):
    agg = parts[0, :_N] + parts[1, :_N]
    agg = agg * diis[:_N, None] + b[None, :]
    mu = jnp.mean(agg, axis=0, keepdims=True)
    var = jnp.mean((agg - mu) ** 2, axis=0, keepdims=True)
    xn = (agg - mu) * lax.rsqrt(var + 1e-5) * g[None, :] + be[None, :]
    return jnp.where(xn > 0, xn, jnp.exp(xn) - 1.0)


def _mid_body(parts_ref, diis_ref, b_ref, g_ref, be_ref, w_ref, dois_ref,
              out_ref):
    e = _bn_elu(parts_ref[...], diis_ref[...], b_ref[...], g_ref[...],
                be_ref[...])
    h = jnp.dot(e, w_ref[...], preferred_element_type=jnp.float32)
    h = h * dois_ref[...][:_N, None]
    out_ref[0:_N, :] = h
    out_ref[_N:_NPAD, :] = jnp.zeros((_NPAD - _N, h.shape[1]), jnp.float32)


def _final_body(parts_ref, diis_ref, b_ref, g_ref, be_ref, ef_ref, wc_ref,
                bc_ref, out_ref):
    x4 = _bn_elu(parts_ref[...], diis_ref[...], b_ref[...], g_ref[...],
                 be_ref[...])
    pooled = jnp.sum(x4, axis=0, keepdims=True)
    colsum = jnp.sum(ef_ref[...], axis=0, keepdims=True)
    em = colsum[:, 0:16]
    for i in range(1, 8):
        em = em + colsum[:, 16 * i:16 * (i + 1)]
    em = em * jnp.float32(1.0 / _E)
    cat = jnp.concatenate([pooled, em], axis=1)
    out_ref[...] = (jnp.dot(cat, wc_ref[...],
                            preferred_element_type=jnp.float32)
                    + bc_ref[...][None, :])


def _f32(*shape):
    return jax.ShapeDtypeStruct(shape, jnp.float32)


def kernel(node_features, edge_index, edge_features, edge_weights,
           W1, b1, gamma1, beta1, W2, b2, gamma2, beta2,
           W3, b3, gamma3, beta3, W4, b4, gamma4, beta4, Wc, bc):
    src = edge_index[0]
    dst = edge_index[1]

    cw, dois, diis = _norm_kernel(src, dst, edge_weights)

    h = pl.pallas_call(_h1_body, out_shape=_f32(_NPAD, 32))(
        node_features, W1, dois)

    layer_params = [(b1, gamma1, beta1, W2), (b2, gamma2, beta2, W3),
                    (b3, gamma3, beta3, W4)]
    for b, g, be, Wn in layer_params:
        aggp = _edge32(h, src, dst, cw)
        h = pl.pallas_call(_mid_body, out_shape=_f32(_NPAD, Wn.shape[1]))(
            aggp, diis, b, g, be, Wn, dois)

    aggp4 = _edge16(h, src, dst, cw)
    ef2 = edge_features.reshape(_E // 8, 128)
    out = pl.pallas_call(_final_body, out_shape=_f32(1, 10))(
        aggp4, diis, b4, gamma4, beta4, ef2, Wc, bc)
    return out


# speedup vs baseline: 1.0664x; 1.0176x over previous
import dataclasses
import functools

import jax
import jax.numpy as jnp
from jax import lax
from jax.experimental import pallas as pl
from jax.experimental.pallas import tpu as pltpu
from jax.experimental.pallas import tpu_sc as plsc

_N = 10000
_NPAD = 10240
_NPART = _NPAD // 16
_E = 320000
_NW = 32
_EPW = _E // _NW
_B = 2000
_NC = _EPW // _B

_mesh = plsc.VectorSubcoreMesh(core_axis_name="c", subcore_axis_name="s")
_CP = pltpu.CompilerParams()
if "needs_layout_passes" in pltpu.CompilerParams.__dataclass_fields__:
    _CP = dataclasses.replace(_CP, needs_layout_passes=False)
_CP = dataclasses.replace(_CP, use_tc_tiling_on_sc=False)


def _rsqrt16(x):
    i = plsc.bitcast(x, jnp.int32)
    i = jnp.int32(0x5F3759DF) - lax.shift_right_logical(i, 1)
    y = plsc.bitcast(i, jnp.float32)
    for _ in range(4):
        y = y * (1.5 - 0.5 * x * y * y)
    return y


_BK = 2000
_RPW = 20000 // _BK


@functools.partial(
    pl.kernel,
    out_type=jax.ShapeDtypeStruct((_E,), jnp.float32),
    mesh=_mesh,
    compiler_params=_CP,
    scratch_types=[
        pltpu.VMEM((_RPW * _BK,), jnp.int32),
        pltpu.VMEM((_RPW * _BK,), jnp.int32),
        pltpu.VMEM((_RPW * _BK,), jnp.float32),
        pltpu.VMEM((_EPW,), jnp.float32),
        pltpu.VMEM((_BK,), jnp.float32),
        pltpu.VMEM((_NPAD,), jnp.float32),
        pltpu.VMEM((_NPAD,), jnp.float32),
        pltpu.VMEM((_NPAD,), jnp.float32),
        pltpu.VMEM((_NPAD,), jnp.float32),
        pltpu.VMEM_SHARED((_NPAD,), jnp.float32),
        pltpu.VMEM_SHARED((_NPAD,), jnp.float32),
        pltpu.VMEM_SHARED((_NPAD,), jnp.float32),
        pltpu.VMEM_SHARED((_NPAD,), jnp.float32),
        pltpu.SemaphoreType.DMA,
        pltpu.SemaphoreType.DMA,
        pltpu.SemaphoreType.DMA,
        pltpu.SemaphoreType.DMA,
    ],
)
def _norm_kernel(src_hbm, dst_hbm, w_hbm, cw_hbm,
                 srcs, dsts, w_v, cw_v, ones_v, ta_v, tb_v, tc_v, td_v,
                 wdo_sh, wdi_sh, cno_sh, cni_sh, m0, m1, m2, m3):
    c = lax.axis_index("c")
    s = lax.axis_index("s")
    wid = s * 2 + c

    pltpu.sync_copy(src_hbm.at[pl.ds(s * _RPW * _BK, _RPW * _BK)], srcs)
    pltpu.sync_copy(dst_hbm.at[pl.ds(s * _RPW * _BK, _RPW * _BK)], dsts)
    pltpu.sync_copy(w_hbm.at[pl.ds(s * _RPW * _BK, _RPW * _BK)], w_v)

    @pl.loop(0, _BK, step=16)
    def _(i):
        ones_v[pl.ds(i, 16)] = jnp.ones((16,), jnp.float32)

    @pl.loop(0, _NPART, step=16)
    def _(i):
        ta_v[pl.ds(i, 16)] = jnp.zeros((16,), jnp.float32)

    for tab in (wdo_sh, wdi_sh, cno_sh, cni_sh):
        pltpu.sync_copy(ta_v.at[pl.ds(0, _NPART)],
                        tab.at[pl.ds(s * _NPART, _NPART)])

    @pl.loop(0, _RPW * _BK, step=16)
    def _(i):
        w_v[pl.ds(i, 16)] = jnp.abs(w_v[pl.ds(i, 16)])

    plsc.subcore_barrier()

    sems = (m0, m1, m2, m3)
    pend = []
    for k in range(_RPW):
        if k >= 2:
            for dsc in pend[0]:
                dsc.wait()
            pend.pop(0)
        wslice = w_v.at[pl.ds(k * _BK, _BK)]
        sslice = srcs.at[pl.ds(k * _BK, _BK)]
        dslice = dsts.at[pl.ds(k * _BK, _BK)]
        pend.append((
            pltpu.async_copy(wslice, wdo_sh.at[sslice], m0, add=True),
            pltpu.async_copy(wslice, wdi_sh.at[dslice], m1, add=True),
            pltpu.async_copy(ones_v, cno_sh.at[sslice], m2, add=True),
            pltpu.async_copy(ones_v, cni_sh.at[dslice], m3, add=True),
        ))
    for group in pend:
        for dsc in group:
            dsc.wait()
    plsc.subcore_barrier()

    pltpu.sync_copy(cno_sh.at[pl.ds(s * _NPART, _NPART)],
                    tc_v.at[pl.ds(0, _NPART)])
    pltpu.sync_copy(cni_sh.at[pl.ds(s * _NPART, _NPART)],
                    td_v.at[pl.ds(0, _NPART)])

    @pl.loop(0, _NPART, step=16)
    def _(i):
        tc_v[pl.ds(i, 16)] = _rsqrt16(jnp.maximum(tc_v[pl.ds(i, 16)], 1.0))
        td_v[pl.ds(i, 16)] = _rsqrt16(jnp.maximum(td_v[pl.ds(i, 16)], 1.0))

    pltpu.sync_copy(tc_v.at[pl.ds(0, _NPART)],
                    cno_sh.at[pl.ds(s * _NPART, _NPART)])
    pltpu.sync_copy(td_v.at[pl.ds(0, _NPART)],
                    cni_sh.at[pl.ds(s * _NPART, _NPART)])
    plsc.subcore_barrier()

    pltpu.sync_copy(wdo_sh, ta_v)
    pltpu.sync_copy(wdi_sh, tb_v)
    pltpu.sync_copy(cno_sh, tc_v)
    pltpu.sync_copy(cni_sh, td_v)
    for r in range(_EPW // _BK):

        @pl.loop(0, _BK // 16)
        def _(g):
            loc = c * _EPW + r * _BK + g * 16
            sv = srcs[pl.ds(loc, 16)]
            dv = dsts[pl.ds(loc, 16)]
            a = plsc.load_gather(ta_v, [sv])
            b = plsc.load_gather(tb_v, [dv])
            do = plsc.load_gather(tc_v, [sv])
            di = plsc.load_gather(td_v, [dv])
            p = jnp.maximum(a * b, jnp.float32(1e-12))
            wv = w_v[pl.ds(loc, 16)]
            cw_v[pl.ds(r * _BK + g * 16, 16)] = (jnp.abs(wv) * _rsqrt16(p)
                                                 * do * di)

    pltpu.sync_copy(cw_v, cw_hbm.at[pl.ds(wid * _EPW, _EPW)])


_BE = 1000
_NCE = _EPW // _BE


def _make_edge_kernel(D):
    @functools.partial(
        pl.kernel,
        out_type=jax.ShapeDtypeStruct((2, _NPAD, D), jnp.float32),
        mesh=_mesh,
        compiler_params=_CP,
        scratch_types=[
            pltpu.VMEM((_EPW,), jnp.int32),
            pltpu.VMEM((_EPW,), jnp.int32),
            pltpu.VMEM((_EPW,), jnp.float32),
            pltpu.VMEM((_BE, D), jnp.float32),
            pltpu.VMEM((_BE, D), jnp.float32),
            pltpu.VMEM_SHARED((_NPAD, D), jnp.float32),
            pltpu.SemaphoreType.DMA,
            pltpu.SemaphoreType.DMA,
            pltpu.SemaphoreType.DMA,
            pltpu.SemaphoreType.DMA,
        ],
    )
    def k(h_hbm, src2_hbm, dst2_hbm, cw_hbm, z_hbm, out_hbm,
          srcs, dsts, cws, r0, r1, agg_sh, g0, g1, q0, q1):
        c = lax.axis_index("c")
        s = lax.axis_index("s")
        wid = s * 2 + c
        rows = (r0, r1)
        gsem = (g0, g1)
        ssem = (q0, q1)

        base = wid * _EPW
        pltpu.sync_copy(src2_hbm.at[pl.ds(base, _EPW)], srcs)
        pltpu.sync_copy(dst2_hbm.at[pl.ds(base, _EPW)], dsts)
        pltpu.sync_copy(cw_hbm.at[pl.ds(base, _EPW)], cws)

        pltpu.sync_copy(z_hbm.at[pl.ds(s * _NPART, _NPART)],
                        agg_sh.at[pl.ds(s * _NPART, _NPART)])
        plsc.subcore_barrier()

        gat = [None, None]
        scat = [None, None]
        gat[0] = pltpu.async_copy(h_hbm.at[srcs.at[pl.ds(0, _BE)]], rows[0],
                                  gsem[0])
        for it in range(_NCE):
            b = it % 2
            nb = 1 - b
            if it + 1 < _NCE:
                if scat[nb] is not None:
                    scat[nb].wait()
                gat[nb] = pltpu.async_copy(
                    h_hbm.at[srcs.at[pl.ds((it + 1) * _BE, _BE)]],
                    rows[nb], gsem[nb])
            gat[b].wait()
            cbase = it * _BE

            @plsc.parallel_loop(0, _BE, step=1, unroll=8)
            def _(e):
                cvec = plsc.load_gather(
                    cws, [jnp.full((16,), cbase, jnp.int32) + e])
                for j in range(0, D, 16):
                    rows[b][e, pl.ds(j, 16)] = rows[b][e, pl.ds(j, 16)] * cvec

            scat[b] = pltpu.async_copy(
                rows[b], agg_sh.at[dsts.at[pl.ds(it * _BE, _BE)]],
                ssem[b], add=True)

        for dsc in scat:
            if dsc is not None:
                dsc.wait()
        plsc.subcore_barrier()
        pltpu.sync_copy(agg_sh.at[pl.ds(s * _NPART, _NPART)],
                        out_hbm.at[c].at[pl.ds(s * _NPART, _NPART)])

    return k


_edge32 = _make_edge_kernel(32)
_edge16 = _make_edge_kernel(16)


def _h1_body(x_ref, w_ref, out_ref):
    h = jnp.dot(x_ref[...], w_ref[...], preferred_element_type=jnp.float32)
    out_ref[0:_N, :] = h
    out_ref[_N:_NPAD, :] = jnp.zeros((_NPAD - _N, h.shape[1]), jnp.float32)


def _bn_elu(parts, b, g, be):
    agg = parts[0, :_N] + parts[1, :_N]
    agg = agg + b[None, :]
    mu = jnp.mean(agg, axis=0, keepdims=True)
    var = jnp.mean((agg - mu) ** 2, axis=0, keepdims=True)
    xn = (agg - mu) * lax.rsqrt(var + 1e-5) * g[None, :] + be[None, :]
    return jnp.where(xn > 0, xn, jnp.exp(xn) - 1.0)


def _mid_body(parts_ref, b_ref, g_ref, be_ref, w_ref, out_ref):
    e = _bn_elu(parts_ref[...], b_ref[...], g_ref[...], be_ref[...])
    h = jnp.dot(e, w_ref[...], preferred_element_type=jnp.float32)
    out_ref[0:_N, :] = h
    out_ref[_N:_NPAD, :] = jnp.zeros((_NPAD - _N, h.shape[1]), jnp.float32)


def _final_body(parts_ref, b_ref, g_ref, be_ref, ef_ref, wc_ref,
                bc_ref, out_ref):
    x4 = _bn_elu(parts_ref[...], b_ref[...], g_ref[...], be_ref[...])
    pooled = jnp.sum(x4, axis=0, keepdims=True)
    colsum = jnp.sum(ef_ref[...], axis=0, keepdims=True)
    em = colsum[:, 0:16]
    for i in range(1, 8):
        em = em + colsum[:, 16 * i:16 * (i + 1)]
    em = em * jnp.float32(1.0 / _E)
    cat = jnp.concatenate([pooled, em], axis=1)
    out_ref[...] = (jnp.dot(cat, wc_ref[...],
                            preferred_element_type=jnp.float32)
                    + bc_ref[...][None, :])


def _f32(*shape):
    return jax.ShapeDtypeStruct(shape, jnp.float32)


def kernel(node_features, edge_index, edge_features, edge_weights,
           W1, b1, gamma1, beta1, W2, b2, gamma2, beta2,
           W3, b3, gamma3, beta3, W4, b4, gamma4, beta4, Wc, bc):
    src = edge_index[0]
    dst = edge_index[1]

    q = _norm_kernel(src, dst, edge_weights)
    z32 = jnp.zeros((_NPAD, 32), jnp.float32)
    z16 = jnp.zeros((_NPAD, 16), jnp.float32)

    h = pl.pallas_call(_h1_body, out_shape=_f32(_NPAD, 32))(
        node_features, W1)

    layer_params = [(b1, gamma1, beta1, W2), (b2, gamma2, beta2, W3),
                    (b3, gamma3, beta3, W4)]
    for b, g, be, Wn in layer_params:
        aggp = _edge32(h, src, dst, q, z32)
        h = pl.pallas_call(_mid_body, out_shape=_f32(_NPAD, Wn.shape[1]))(
            aggp, b, g, be, Wn)

    aggp4 = _edge16(h, src, dst, q, z16)
    ef2 = edge_features.reshape(_E // 8, 128)
    out = pl.pallas_call(_final_body, out_shape=_f32(1, 10))(
        aggp4, b4, gamma4, beta4, ef2, Wc, bc)
    return out


# speedup vs baseline: 1.0907x; 1.0228x over previous
import dataclasses
import functools

import jax
import jax.numpy as jnp
from jax import lax
from jax.experimental import pallas as pl
from jax.experimental.pallas import tpu as pltpu
from jax.experimental.pallas import tpu_sc as plsc

_N = 10000
_NPAD = 10240
_NPART = _NPAD // 16
_E = 320000
_NW = 32
_EPW = _E // _NW
_B = 2000
_NC = _EPW // _B

_mesh = plsc.VectorSubcoreMesh(core_axis_name="c", subcore_axis_name="s")
_CP = pltpu.CompilerParams()
if "needs_layout_passes" in pltpu.CompilerParams.__dataclass_fields__:
    _CP = dataclasses.replace(_CP, needs_layout_passes=False)
_CP = dataclasses.replace(_CP, use_tc_tiling_on_sc=False)


def _rsqrt16(x):
    i = plsc.bitcast(x, jnp.int32)
    i = jnp.int32(0x5F3759DF) - lax.shift_right_logical(i, 1)
    y = plsc.bitcast(i, jnp.float32)
    for _ in range(4):
        y = y * (1.5 - 0.5 * x * y * y)
    return y


_BK = 2000
_RPW = 20000 // _BK


@functools.partial(
    pl.kernel,
    out_type=jax.ShapeDtypeStruct((_E,), jnp.float32),
    mesh=_mesh,
    compiler_params=_CP,
    scratch_types=[
        pltpu.VMEM((_RPW * _BK,), jnp.int32),
        pltpu.VMEM((_RPW * _BK,), jnp.int32),
        pltpu.VMEM((_RPW * _BK,), jnp.float32),
        pltpu.VMEM((_EPW,), jnp.float32),
        pltpu.VMEM((_BK,), jnp.float32),
        pltpu.VMEM((_NPAD,), jnp.float32),
        pltpu.VMEM((_NPAD,), jnp.float32),
        pltpu.VMEM((_NPAD,), jnp.float32),
        pltpu.VMEM((_NPAD,), jnp.float32),
        pltpu.VMEM_SHARED((_NPAD,), jnp.float32),
        pltpu.VMEM_SHARED((_NPAD,), jnp.float32),
        pltpu.VMEM_SHARED((_NPAD,), jnp.float32),
        pltpu.VMEM_SHARED((_NPAD,), jnp.float32),
        pltpu.SemaphoreType.DMA,
        pltpu.SemaphoreType.DMA,
        pltpu.SemaphoreType.DMA,
        pltpu.SemaphoreType.DMA,
    ],
)
def _norm_kernel(ei_hbm, w_hbm, cw_hbm,
                 srcs, dsts, w_v, cw_v, ones_v, ta_v, tb_v, tc_v, td_v,
                 wdo_sh, wdi_sh, cno_sh, cni_sh, m0, m1, m2, m3):
    c = lax.axis_index("c")
    s = lax.axis_index("s")
    wid = s * 2 + c

    pltpu.sync_copy(ei_hbm.at[0].at[pl.ds(s * _RPW * _BK, _RPW * _BK)], srcs)
    pltpu.sync_copy(ei_hbm.at[1].at[pl.ds(s * _RPW * _BK, _RPW * _BK)], dsts)
    pltpu.sync_copy(w_hbm.at[pl.ds(s * _RPW * _BK, _RPW * _BK)], w_v)

    @pl.loop(0, _BK, step=16)
    def _(i):
        ones_v[pl.ds(i, 16)] = jnp.ones((16,), jnp.float32)

    @pl.loop(0, _NPART, step=16)
    def _(i):
        ta_v[pl.ds(i, 16)] = jnp.zeros((16,), jnp.float32)

    for tab in (wdo_sh, wdi_sh, cno_sh, cni_sh):
        pltpu.sync_copy(ta_v.at[pl.ds(0, _NPART)],
                        tab.at[pl.ds(s * _NPART, _NPART)])

    @pl.loop(0, _RPW * _BK, step=16)
    def _(i):
        w_v[pl.ds(i, 16)] = jnp.abs(w_v[pl.ds(i, 16)])

    plsc.subcore_barrier()

    sems = (m0, m1, m2, m3)
    pend = []
    for k in range(_RPW):
        if k >= 2:
            for dsc in pend[0]:
                dsc.wait()
            pend.pop(0)
        wslice = w_v.at[pl.ds(k * _BK, _BK)]
        sslice = srcs.at[pl.ds(k * _BK, _BK)]
        dslice = dsts.at[pl.ds(k * _BK, _BK)]
        pend.append((
            pltpu.async_copy(wslice, wdo_sh.at[sslice], m0, add=True),
            pltpu.async_copy(wslice, wdi_sh.at[dslice], m1, add=True),
            pltpu.async_copy(ones_v, cno_sh.at[sslice], m2, add=True),
            pltpu.async_copy(ones_v, cni_sh.at[dslice], m3, add=True),
        ))
    for group in pend:
        for dsc in group:
            dsc.wait()
    plsc.subcore_barrier()

    pltpu.sync_copy(cno_sh.at[pl.ds(s * _NPART, _NPART)],
                    tc_v.at[pl.ds(0, _NPART)])
    pltpu.sync_copy(cni_sh.at[pl.ds(s * _NPART, _NPART)],
                    td_v.at[pl.ds(0, _NPART)])

    @pl.loop(0, _NPART, step=16)
    def _(i):
        tc_v[pl.ds(i, 16)] = _rsqrt16(jnp.maximum(tc_v[pl.ds(i, 16)], 1.0))
        td_v[pl.ds(i, 16)] = _rsqrt16(jnp.maximum(td_v[pl.ds(i, 16)], 1.0))

    pltpu.sync_copy(tc_v.at[pl.ds(0, _NPART)],
                    cno_sh.at[pl.ds(s * _NPART, _NPART)])
    pltpu.sync_copy(td_v.at[pl.ds(0, _NPART)],
                    cni_sh.at[pl.ds(s * _NPART, _NPART)])
    plsc.subcore_barrier()

    pltpu.sync_copy(wdo_sh, ta_v)
    pltpu.sync_copy(wdi_sh, tb_v)
    pltpu.sync_copy(cno_sh, tc_v)
    pltpu.sync_copy(cni_sh, td_v)
    for r in range(_EPW // _BK):

        @pl.loop(0, _BK // 16)
        def _(g):
            loc = c * _EPW + r * _BK + g * 16
            sv = srcs[pl.ds(loc, 16)]
            dv = dsts[pl.ds(loc, 16)]
            a = plsc.load_gather(ta_v, [sv])
            b = plsc.load_gather(tb_v, [dv])
            do = plsc.load_gather(tc_v, [sv])
            di = plsc.load_gather(td_v, [dv])
            p = jnp.maximum(a * b, jnp.float32(1e-12))
            wv = w_v[pl.ds(loc, 16)]
            cw_v[pl.ds(r * _BK + g * 16, 16)] = (jnp.abs(wv) * _rsqrt16(p)
                                                 * do * di)

    pltpu.sync_copy(cw_v, cw_hbm.at[pl.ds(wid * _EPW, _EPW)])


_BE = 1000
_NCE = _EPW // _BE


def _make_edge_kernel(D):
    @functools.partial(
        pl.kernel,
        out_type=jax.ShapeDtypeStruct((2, _NPAD, D), jnp.float32),
        mesh=_mesh,
        compiler_params=_CP,
        scratch_types=[
            pltpu.VMEM((_EPW,), jnp.int32),
            pltpu.VMEM((_EPW,), jnp.int32),
            pltpu.VMEM((_EPW,), jnp.float32),
            pltpu.VMEM((_BE, D), jnp.float32),
            pltpu.VMEM((_BE, D), jnp.float32),
            pltpu.VMEM_SHARED((_NPAD, D), jnp.float32),
            pltpu.SemaphoreType.DMA,
            pltpu.SemaphoreType.DMA,
            pltpu.SemaphoreType.DMA,
            pltpu.SemaphoreType.DMA,
        ],
    )
    def k(h_hbm, ei_hbm, cw_hbm, z_hbm, out_hbm,
          srcs, dsts, cws, r0, r1, agg_sh, g0, g1, q0, q1):
        c = lax.axis_index("c")
        s = lax.axis_index("s")
        wid = s * 2 + c
        rows = (r0, r1)
        gsem = (g0, g1)
        ssem = (q0, q1)

        base = wid * _EPW
        pltpu.sync_copy(ei_hbm.at[0].at[pl.ds(base, _EPW)], srcs)
        pltpu.sync_copy(ei_hbm.at[1].at[pl.ds(base, _EPW)], dsts)
        pltpu.sync_copy(cw_hbm.at[pl.ds(base, _EPW)], cws)

        pltpu.sync_copy(z_hbm.at[pl.ds(s * _NPART, _NPART)],
                        agg_sh.at[pl.ds(s * _NPART, _NPART)])
        plsc.subcore_barrier()

        gat = [None, None]
        scat = [None, None]
        gat[0] = pltpu.async_copy(h_hbm.at[srcs.at[pl.ds(0, _BE)]], rows[0],
                                  gsem[0])
        for it in range(_NCE):
            b = it % 2
            nb = 1 - b
            if it + 1 < _NCE:
                if scat[nb] is not None:
                    scat[nb].wait()
                gat[nb] = pltpu.async_copy(
                    h_hbm.at[srcs.at[pl.ds((it + 1) * _BE, _BE)]],
                    rows[nb], gsem[nb])
            gat[b].wait()
            cbase = it * _BE

            @plsc.parallel_loop(0, _BE, step=1, unroll=8)
            def _(e):
                cvec = plsc.load_gather(
                    cws, [jnp.full((16,), cbase, jnp.int32) + e])
                for j in range(0, D, 16):
                    rows[b][e, pl.ds(j, 16)] = rows[b][e, pl.ds(j, 16)] * cvec

            scat[b] = pltpu.async_copy(
                rows[b], agg_sh.at[dsts.at[pl.ds(it * _BE, _BE)]],
                ssem[b], add=True)

        for dsc in scat:
            if dsc is not None:
                dsc.wait()
        plsc.subcore_barrier()
        pltpu.sync_copy(agg_sh.at[pl.ds(s * _NPART, _NPART)],
                        out_hbm.at[c].at[pl.ds(s * _NPART, _NPART)])

    return k


_edge32 = _make_edge_kernel(32)
_edge16 = _make_edge_kernel(16)


def _h1_body(x_ref, w_ref, out_ref):
    h = jnp.dot(x_ref[...], w_ref[...], preferred_element_type=jnp.float32)
    out_ref[0:_N, :] = h
    out_ref[_N:_NPAD, :] = jnp.zeros((_NPAD - _N, h.shape[1]), jnp.float32)


def _bn_elu(parts, b, g, be):
    agg = parts[0, :_N] + parts[1, :_N]
    agg = agg + b[None, :]
    mu = jnp.mean(agg, axis=0, keepdims=True)
    var = jnp.mean((agg - mu) ** 2, axis=0, keepdims=True)
    xn = (agg - mu) * lax.rsqrt(var + 1e-5) * g[None, :] + be[None, :]
    return jnp.where(xn > 0, xn, jnp.exp(xn) - 1.0)


def _mid_body(parts_ref, b_ref, g_ref, be_ref, w_ref, out_ref):
    e = _bn_elu(parts_ref[...], b_ref[...], g_ref[...], be_ref[...])
    h = jnp.dot(e, w_ref[...], preferred_element_type=jnp.float32)
    out_ref[0:_N, :] = h
    out_ref[_N:_NPAD, :] = jnp.zeros((_NPAD - _N, h.shape[1]), jnp.float32)


def _efsum_body(ef_ref, out_ref):
    i = pl.program_id(0)

    @pl.when(i == 0)
    def _():
        out_ref[...] = jnp.zeros_like(out_ref)

    out_ref[...] += jnp.sum(ef_ref[...], axis=0, keepdims=True)


def _final_body(parts_ref, b_ref, g_ref, be_ref, em_ref, wc_ref,
                bc_ref, out_ref):
    x4 = _bn_elu(parts_ref[...], b_ref[...], g_ref[...], be_ref[...])
    pooled = jnp.sum(x4, axis=0, keepdims=True)
    em = em_ref[...] * jnp.float32(1.0 / _E)
    cat = jnp.concatenate([pooled, em], axis=1)
    out_ref[...] = (jnp.dot(cat, wc_ref[...],
                            preferred_element_type=jnp.float32)
                    + bc_ref[...][None, :])


def _f32(*shape):
    return jax.ShapeDtypeStruct(shape, jnp.float32)


def kernel(node_features, edge_index, edge_features, edge_weights,
           W1, b1, gamma1, beta1, W2, b2, gamma2, beta2,
           W3, b3, gamma3, beta3, W4, b4, gamma4, beta4, Wc, bc):
    q = _norm_kernel(edge_index, edge_weights)
    efsum = pl.pallas_call(
        _efsum_body,
        grid=(40,),
        in_specs=[pl.BlockSpec((_E // 40, 16), lambda i: (i, 0))],
        out_specs=pl.BlockSpec((1, 16), lambda i: (0, 0)),
        out_shape=_f32(1, 16),
    )(edge_features)
    q, efsum = lax.optimization_barrier((q, efsum))
    z32 = jnp.zeros((_NPAD, 32), jnp.float32)
    z16 = jnp.zeros((_NPAD, 16), jnp.float32)

    h = pl.pallas_call(_h1_body, out_shape=_f32(_NPAD, 32))(
        node_features, W1)

    layer_params = [(b1, gamma1, beta1, W2), (b2, gamma2, beta2, W3),
                    (b3, gamma3, beta3, W4)]
    for b, g, be, Wn in layer_params:
        aggp = _edge32(h, edge_index, q, z32)
        h = pl.pallas_call(_mid_body, out_shape=_f32(_NPAD, Wn.shape[1]))(
            aggp, b, g, be, Wn)

    aggp4 = _edge16(h, edge_index, q, z16)
    out = pl.pallas_call(_final_body, out_shape=_f32(1, 10))(
        aggp4, b4, gamma4, beta4, efsum, Wc, bc)
    return out
